# baseline (speedup 1.0000x reference)
import functools

import jax
import jax.numpy as jnp
from jax import lax
from jax.experimental import pallas as pl
from jax.experimental.pallas import tpu as pltpu
from jax.experimental.pallas import tpu_sc as plsc

N = 10000
E = 320000
FEAT = 128
NCLASS = 40

NC = 2
NS = 16
NLANE = 16
NW = NC * NS
CHUNK = 128
NCH0 = 120
NCH1 = 40
E_TOT = NS * (NCH0 + NCH1) * CHUNK
E_PAD = E_TOT - E
N_PAD = 10240
RPT = N_PAD // NS
CROWS = N_PAD // FEAT
NBUF = 2

_mesh = plsc.VectorSubcoreMesh(core_axis_name="c", subcore_axis_name="s")


@functools.partial(
    pl.kernel,
    out_type=jax.ShapeDtypeStruct((NC, N_PAD, FEAT), jnp.float32),
    mesh=_mesh,
    scratch_types=[
        pltpu.VMEM((NCH0, CHUNK), jnp.int32),
        pltpu.VMEM((NBUF, 1, CHUNK), jnp.int32),
        pltpu.VMEM((NBUF, CHUNK, FEAT), jnp.float32),
        pltpu.VMEM_SHARED((N_PAD, FEAT), jnp.float32),
        pltpu.SemaphoreType.DMA,
        pltpu.SemaphoreType.DMA,
        pltpu.SemaphoreType.DMA,
        pltpu.SemaphoreType.DMA,
    ],
)
def _sc_agg(x_hbm, src_hbm, dst_hbm, sums_out,
            src_v, dst_buf, rows_v, acc, gsem0, gsem1, dsem0, dsem1):
    c = lax.axis_index("c")
    s = lax.axis_index("s")
    wid = c * NS + s
    nch = jnp.where(c == 0, NCH0, NCH1)
    gsems = (gsem0, gsem1)
    dsems = (dsem0, dsem1)

    pltpu.sync_copy(src_hbm.at[wid], src_v)

    zeros16 = jnp.zeros((NLANE,), jnp.float32)

    @pl.loop(0, CHUNK)
    def _(i):
        for j in range(FEAT // NLANE):
            rows_v[0, i, pl.ds(j * NLANE, NLANE)] = zeros16

    base = s * RPT
    for k in range(RPT // CHUNK):
        pltpu.sync_copy(rows_v.at[0], acc.at[pl.ds(base + k * CHUNK, CHUNK)])

    for b in range(NBUF):
        pltpu.async_copy(dst_hbm.at[wid, b], dst_buf.at[b], dsems[b])
        pltpu.async_copy(x_hbm.at[src_v.at[b]], rows_v.at[b], gsems[b])

    plsc.subcore_barrier()

    @pl.loop(0, nch, step=NBUF)
    def _(i0):
        for b in range(NBUF):
            i = i0 + b
            pltpu.make_async_copy(
                dst_hbm.at[wid, i], dst_buf.at[b], dsems[b]).wait()
            pltpu.make_async_copy(
                x_hbm.at[src_v.at[i]], rows_v.at[b], gsems[b]).wait()
            pltpu.sync_copy(rows_v.at[b], acc.at[dst_buf.at[b, 0]], add=True)
            nxt = i + NBUF

            @pl.when(nxt < nch)
            def _():
                pltpu.async_copy(dst_hbm.at[wid, nxt], dst_buf.at[b], dsems[b])
                pltpu.async_copy(
                    x_hbm.at[src_v.at[nxt]], rows_v.at[b], gsems[b])

    plsc.subcore_barrier()
    pltpu.sync_copy(acc.at[pl.ds(base, RPT)], sums_out.at[c, pl.ds(base, RPT)])


BLK = 1024


def _mean_agg(sums_ref, cnt_ref):
    cnt = cnt_ref[0] + cnt_ref[1]
    return (sums_ref[0] + sums_ref[1]) / jnp.maximum(cnt, 1.0)


def _l2_normalize(h):
    nrm = jnp.sqrt(jnp.sum(h * h, axis=1, keepdims=True))
    return h / jnp.maximum(nrm, 1e-12)


def _dense1_body(sums_ref, cnt_ref, x_ref, wl_ref, bl_ref, wr_ref, o_ref):
    agg = _mean_agg(sums_ref, cnt_ref)
    h = (jnp.dot(agg, wl_ref[...], preferred_element_type=jnp.float32)
         + jnp.dot(x_ref[...], wr_ref[...], preferred_element_type=jnp.float32)
         + bl_ref[...])
    o_ref[...] = jnp.maximum(_l2_normalize(h), 0.0)


def _dense2_body(sums_ref, cnt_ref, h_ref, wl_ref, bl_ref, wr_ref,
                 wfc_ref, bfc_ref, o_ref):
    agg = _mean_agg(sums_ref, cnt_ref)
    h2 = (jnp.dot(agg, wl_ref[...], preferred_element_type=jnp.float32)
          + jnp.dot(h_ref[...], wr_ref[...], preferred_element_type=jnp.float32)
          + bl_ref[...])
    h2 = _l2_normalize(h2)
    logits = jnp.dot(h2, wfc_ref[...], preferred_element_type=jnp.float32)
    logits = logits + bfc_ref[...]
    m = jnp.max(logits, axis=1, keepdims=True)
    e = jnp.exp(logits - m)
    o_ref[...] = e / jnp.sum(e, axis=1, keepdims=True)


_full = pl.BlockSpec((FEAT, FEAT), lambda i: (0, 0))
_bias = pl.BlockSpec((1, FEAT), lambda i: (0, 0))
_rows = pl.BlockSpec((BLK, FEAT), lambda i: (i, 0))
_sums = pl.BlockSpec((NC, BLK, FEAT), lambda i: (0, i, 0))
_cnts = _sums

_dense1 = pl.pallas_call(
    _dense1_body,
    grid=(N_PAD // BLK,),
    in_specs=[_sums, _cnts, _rows, _full, _bias, _full],
    out_specs=_rows,
    out_shape=jax.ShapeDtypeStruct((N_PAD, FEAT), jnp.float32),
)

_dense2 = pl.pallas_call(
    _dense2_body,
    grid=(N_PAD // BLK,),
    in_specs=[_sums, _cnts, _rows, _full, _bias, _full, _full, _bias],
    out_specs=_rows,
    out_shape=jax.ShapeDtypeStruct((N_PAD, FEAT), jnp.float32),
)


@jax.jit
def kernel(x, edge_index, W1l, b1l, W1r, W2l, b2l, W2r, Wfc, bfc):
    x_pad = jnp.pad(x, ((0, N_PAD - N), (0, 0)))
    pad_src = jnp.zeros((E_PAD,), jnp.int32)
    pad_dst = N + (jnp.arange(E_PAD, dtype=jnp.int32) % (N_PAD - N))
    src_all = jnp.concatenate([edge_index[0], pad_src])
    dst_all = jnp.concatenate([edge_index[1], pad_dst])
    ec0 = NS * NCH0 * CHUNK
    s0 = src_all[:ec0].reshape(NS, NCH0, CHUNK)
    s1 = jnp.pad(src_all[ec0:].reshape(NS, NCH1, CHUNK),
                 ((0, 0), (0, NCH0 - NCH1), (0, 0)))
    src3 = jnp.concatenate([s0, s1], axis=0)
    d0 = dst_all[:ec0].reshape(NS, NCH0, 1, CHUNK)
    d1 = jnp.pad(dst_all[ec0:].reshape(NS, NCH1, 1, CHUNK),
                 ((0, 0), (0, NCH0 - NCH1), (0, 0), (0, 0)))
    dst4 = jnp.concatenate([d0, d1], axis=0)

    ones_mat = jnp.ones((N_PAD, FEAT), jnp.float32)
    cnt = _sc_agg(ones_mat, src3, dst4)
    sums1 = _sc_agg(x_pad, src3, dst4)
    h1 = _dense1(sums1, cnt, x_pad, W1l.T, b1l.reshape(1, FEAT), W1r.T)

    sums2 = _sc_agg(h1, src3, dst4)
    wfc_pad = jnp.zeros((FEAT, FEAT), jnp.float32).at[:, :NCLASS].set(Wfc.T)
    bfc_pad = jnp.full((1, FEAT), -1e30, jnp.float32).at[0, :NCLASS].set(bfc)
    probs = _dense2(sums2, cnt, h1,
                    W2l.T, b2l.reshape(1, FEAT), W2r.T, wfc_pad, bfc_pad)
    return probs[:N, :NCLASS]


# speedup vs baseline: 3.4157x; 3.4157x over previous
import functools

import jax
import jax.numpy as jnp
from jax import lax
from jax.experimental import pallas as pl
from jax.experimental.pallas import tpu as pltpu
from jax.experimental.pallas import tpu_sc as plsc

N = 10000
E = 320000
FEAT = 128
NCLASS = 40

NC = 2
NS = 16
NLANE = 16
NW = NC * NS
CHUNK = 128
NCH0 = 80
NCH1 = 80
E_TOT = NS * (NCH0 + NCH1) * CHUNK
E_PAD = E_TOT - E
N_PAD = 10240
RPT = N_PAD // NS
CROWS = N_PAD // FEAT
NBUF = 2

_mesh = plsc.VectorSubcoreMesh(core_axis_name="c", subcore_axis_name="s")


@functools.partial(
    pl.kernel,
    out_type=jax.ShapeDtypeStruct((NC, N_PAD, FEAT), jnp.float32),
    mesh=_mesh,
    scratch_types=[
        pltpu.VMEM((NCH0, CHUNK), jnp.int32),
        pltpu.VMEM((NBUF, 1, CHUNK), jnp.int32),
        pltpu.VMEM((NBUF, CHUNK, FEAT), jnp.float32),
        pltpu.VMEM_SHARED((N_PAD, FEAT), jnp.float32),
        pltpu.SemaphoreType.DMA,
        pltpu.SemaphoreType.DMA,
        pltpu.SemaphoreType.DMA,
        pltpu.SemaphoreType.DMA,
    ],
)
def _sc_agg(x_hbm, src_hbm, dst_hbm, sums_out,
            src_v, dst_buf, rows_v, acc, gsem0, gsem1, dsem0, dsem1):
    c = lax.axis_index("c")
    s = lax.axis_index("s")
    wid = c * NS + s
    nch = jnp.where(c == 0, NCH0, NCH1)
    gsems = (gsem0, gsem1)
    dsems = (dsem0, dsem1)

    pltpu.sync_copy(src_hbm.at[wid], src_v)

    zeros16 = jnp.zeros((NLANE,), jnp.float32)

    @pl.loop(0, CHUNK)
    def _(i):
        for j in range(FEAT // NLANE):
            rows_v[0, i, pl.ds(j * NLANE, NLANE)] = zeros16

    base = s * RPT
    for k in range(RPT // CHUNK):
        pltpu.sync_copy(rows_v.at[0], acc.at[pl.ds(base + k * CHUNK, CHUNK)])

    for b in range(NBUF):
        pltpu.async_copy(dst_hbm.at[wid, b], dst_buf.at[b], dsems[b])
        pltpu.async_copy(x_hbm.at[src_v.at[b]], rows_v.at[b], gsems[b])

    plsc.subcore_barrier()

    @pl.loop(0, nch, step=NBUF)
    def _(i0):
        for b in range(NBUF):
            i = i0 + b
            pltpu.make_async_copy(
                dst_hbm.at[wid, i], dst_buf.at[b], dsems[b]).wait()
            pltpu.make_async_copy(
                x_hbm.at[src_v.at[i]], rows_v.at[b], gsems[b]).wait()
            pltpu.sync_copy(rows_v.at[b], acc.at[dst_buf.at[b, 0]], add=True)
            nxt = i + NBUF

            @pl.when(nxt < nch)
            def _():
                pltpu.async_copy(dst_hbm.at[wid, nxt], dst_buf.at[b], dsems[b])
                pltpu.async_copy(
                    x_hbm.at[src_v.at[nxt]], rows_v.at[b], gsems[b])

    plsc.subcore_barrier()
    pltpu.sync_copy(acc.at[pl.ds(base, RPT)], sums_out.at[c, pl.ds(base, RPT)])


BLK = 1024


def _mean_agg(sums_ref, cnt_ref):
    cnt = cnt_ref[0] + cnt_ref[1]
    return (sums_ref[0] + sums_ref[1]) / jnp.maximum(cnt, 1.0)


def _l2_normalize(h):
    nrm = jnp.sqrt(jnp.sum(h * h, axis=1, keepdims=True))
    return h / jnp.maximum(nrm, 1e-12)


def _dense1_body(sums_ref, cnt_ref, x_ref, wl_ref, bl_ref, wr_ref, o_ref):
    agg = _mean_agg(sums_ref, cnt_ref)
    h = (jnp.dot(agg, wl_ref[...], preferred_element_type=jnp.float32)
         + jnp.dot(x_ref[...], wr_ref[...], preferred_element_type=jnp.float32)
         + bl_ref[...])
    o_ref[...] = jnp.maximum(_l2_normalize(h), 0.0)


def _dense2_body(sums_ref, cnt_ref, h_ref, wl_ref, bl_ref, wr_ref,
                 wfc_ref, bfc_ref, o_ref):
    agg = _mean_agg(sums_ref, cnt_ref)
    h2 = (jnp.dot(agg, wl_ref[...], preferred_element_type=jnp.float32)
          + jnp.dot(h_ref[...], wr_ref[...], preferred_element_type=jnp.float32)
          + bl_ref[...])
    h2 = _l2_normalize(h2)
    logits = jnp.dot(h2, wfc_ref[...], preferred_element_type=jnp.float32)
    logits = logits + bfc_ref[...]
    m = jnp.max(logits, axis=1, keepdims=True)
    e = jnp.exp(logits - m)
    o_ref[...] = e / jnp.sum(e, axis=1, keepdims=True)


_full = pl.BlockSpec((FEAT, FEAT), lambda i: (0, 0))
_bias = pl.BlockSpec((1, FEAT), lambda i: (0, 0))
_rows = pl.BlockSpec((BLK, FEAT), lambda i: (i, 0))
_sums = pl.BlockSpec((NC, BLK, FEAT), lambda i: (0, i, 0))
_cnts = _sums

_dense1 = pl.pallas_call(
    _dense1_body,
    grid=(N_PAD // BLK,),
    in_specs=[_sums, _cnts, _rows, _full, _bias, _full],
    out_specs=_rows,
    out_shape=jax.ShapeDtypeStruct((N_PAD, FEAT), jnp.float32),
)

_dense2 = pl.pallas_call(
    _dense2_body,
    grid=(N_PAD // BLK,),
    in_specs=[_sums, _cnts, _rows, _full, _bias, _full, _full, _bias],
    out_specs=_rows,
    out_shape=jax.ShapeDtypeStruct((N_PAD, FEAT), jnp.float32),
)


@jax.jit
def kernel(x, edge_index, W1l, b1l, W1r, W2l, b2l, W2r, Wfc, bfc):
    x_pad = jnp.pad(x, ((0, N_PAD - N), (0, 0)))
    pad_src = jnp.arange(E_PAD, dtype=jnp.int32) % N
    pad_dst = N + (jnp.arange(E_PAD, dtype=jnp.int32) % (N_PAD - N))
    src_all = jnp.concatenate([edge_index[0], pad_src])
    dst_all = jnp.concatenate([edge_index[1], pad_dst])
    ec0 = NS * NCH0 * CHUNK
    s0 = src_all[:ec0].reshape(NS, NCH0, CHUNK)
    s1 = jnp.pad(src_all[ec0:].reshape(NS, NCH1, CHUNK),
                 ((0, 0), (0, NCH0 - NCH1), (0, 0)))
    src3 = jnp.concatenate([s0, s1], axis=0)
    d0 = dst_all[:ec0].reshape(NS, NCH0, 1, CHUNK)
    d1 = jnp.pad(dst_all[ec0:].reshape(NS, NCH1, 1, CHUNK),
                 ((0, 0), (0, NCH0 - NCH1), (0, 0), (0, 0)))
    dst4 = jnp.concatenate([d0, d1], axis=0)

    ones_mat = jnp.ones((N_PAD, FEAT), jnp.float32)
    cnt = _sc_agg(ones_mat, src3, dst4)
    sums1 = _sc_agg(x_pad, src3, dst4)
    h1 = _dense1(sums1, cnt, x_pad, W1l.T, b1l.reshape(1, FEAT), W1r.T)

    sums2 = _sc_agg(h1, src3, dst4)
    wfc_pad = jnp.zeros((FEAT, FEAT), jnp.float32).at[:, :NCLASS].set(Wfc.T)
    bfc_pad = jnp.full((1, FEAT), -1e30, jnp.float32).at[0, :NCLASS].set(bfc)
    probs = _dense2(sums2, cnt, h1,
                    W2l.T, b2l.reshape(1, FEAT), W2r.T, wfc_pad, bfc_pad)
    return probs[:N, :NCLASS]


# speedup vs baseline: 3.7587x; 1.1004x over previous
import functools

import jax
import jax.numpy as jnp
from jax import lax
from jax.experimental import pallas as pl
from jax.experimental.pallas import tpu as pltpu
from jax.experimental.pallas import tpu_sc as plsc

N = 10000
E = 320000
FEAT = 128
NCLASS = 40

NC = 2
NS = 16
NLANE = 16
NW = NC * NS
CHUNK = 128
NCH0 = 80
NCH1 = 80
E_TOT = NS * (NCH0 + NCH1) * CHUNK
E_PAD = E_TOT - E
N_PAD = 10240
RPT = N_PAD // NS
CROWS = N_PAD // FEAT
NBUF = 2

_mesh = plsc.VectorSubcoreMesh(core_axis_name="c", subcore_axis_name="s")


@functools.partial(
    pl.kernel,
    out_type=jax.ShapeDtypeStruct((NC, N_PAD, FEAT), jnp.float32),
    mesh=_mesh,
    scratch_types=[
        pltpu.VMEM((NCH0, CHUNK), jnp.int32),
        pltpu.VMEM((NBUF, 1, CHUNK), jnp.int32),
        pltpu.VMEM((NBUF, CHUNK, FEAT), jnp.float32),
        pltpu.VMEM_SHARED((N_PAD, FEAT), jnp.float32),
        pltpu.SemaphoreType.DMA,
        pltpu.SemaphoreType.DMA,
        pltpu.SemaphoreType.DMA,
        pltpu.SemaphoreType.DMA,
    ],
)
def _sc_agg(x_hbm, src_hbm, dst_hbm, sums_out,
            src_v, dst_buf, rows_v, acc, gsem0, gsem1, dsem0, dsem1):
    c = lax.axis_index("c")
    s = lax.axis_index("s")
    wid = c * NS + s
    nch = jnp.where(c == 0, NCH0, NCH1)
    gsems = (gsem0, gsem1)
    dsems = (dsem0, dsem1)

    pltpu.sync_copy(src_hbm.at[wid], src_v)

    zeros16 = jnp.zeros((NLANE,), jnp.float32)

    @pl.loop(0, CHUNK)
    def _(i):
        for j in range(FEAT // NLANE):
            rows_v[0, i, pl.ds(j * NLANE, NLANE)] = zeros16

    base = s * RPT
    for k in range(RPT // CHUNK):
        pltpu.sync_copy(rows_v.at[0], acc.at[pl.ds(base + k * CHUNK, CHUNK)])

    for b in range(NBUF):
        pltpu.async_copy(dst_hbm.at[wid, b], dst_buf.at[b], dsems[b])
        pltpu.async_copy(x_hbm.at[src_v.at[b]], rows_v.at[b], gsems[b])

    plsc.subcore_barrier()

    @pl.loop(0, nch, step=NBUF)
    def _(i0):
        for b in range(NBUF):
            i = i0 + b
            pltpu.make_async_copy(
                dst_hbm.at[wid, i], dst_buf.at[b], dsems[b]).wait()
            pltpu.make_async_copy(
                x_hbm.at[src_v.at[i]], rows_v.at[b], gsems[b]).wait()
            pltpu.sync_copy(rows_v.at[b], acc.at[dst_buf.at[b, 0]], add=True)
            nxt = i + NBUF

            @pl.when(nxt < nch)
            def _():
                pltpu.async_copy(dst_hbm.at[wid, nxt], dst_buf.at[b], dsems[b])
                pltpu.async_copy(
                    x_hbm.at[src_v.at[nxt]], rows_v.at[b], gsems[b])

    plsc.subcore_barrier()
    pltpu.sync_copy(acc.at[pl.ds(base, RPT)], sums_out.at[c, pl.ds(base, RPT)])


@functools.partial(
    pl.kernel,
    out_type=jax.ShapeDtypeStruct((NC, N_PAD, FEAT), jnp.float32),
    mesh=_mesh,
    scratch_types=[
        pltpu.VMEM((NBUF, 1, CHUNK), jnp.int32),
        pltpu.VMEM((CHUNK, FEAT), jnp.float32),
        pltpu.VMEM_SHARED((N_PAD, FEAT), jnp.float32),
        pltpu.SemaphoreType.DMA,
        pltpu.SemaphoreType.DMA,
    ],
)
def _sc_counts(dst_hbm, cnt_out, dst_buf, ones_v, acc, dsem0, dsem1):
    c = lax.axis_index("c")
    s = lax.axis_index("s")
    wid = c * NS + s
    dsems = (dsem0, dsem1)

    for b in range(NBUF):
        pltpu.async_copy(dst_hbm.at[wid, b], dst_buf.at[b], dsems[b])

    zeros16 = jnp.zeros((NLANE,), jnp.float32)
    ones16 = jnp.ones((NLANE,), jnp.float32)

    @pl.loop(0, CHUNK)
    def _(i):
        for j in range(FEAT // NLANE):
            ones_v[i, pl.ds(j * NLANE, NLANE)] = zeros16

    base = s * RPT
    for k in range(RPT // CHUNK):
        pltpu.sync_copy(ones_v, acc.at[pl.ds(base + k * CHUNK, CHUNK)])

    @pl.loop(0, CHUNK)
    def _(i):
        for j in range(FEAT // NLANE):
            ones_v[i, pl.ds(j * NLANE, NLANE)] = ones16

    plsc.subcore_barrier()

    @pl.loop(0, NCH0, step=NBUF)
    def _(i0):
        for b in range(NBUF):
            i = i0 + b
            pltpu.make_async_copy(
                dst_hbm.at[wid, i], dst_buf.at[b], dsems[b]).wait()
            pltpu.sync_copy(ones_v, acc.at[dst_buf.at[b, 0]], add=True)
            nxt = i + NBUF

            @pl.when(nxt < NCH0)
            def _():
                pltpu.async_copy(dst_hbm.at[wid, nxt], dst_buf.at[b], dsems[b])

    plsc.subcore_barrier()
    pltpu.sync_copy(acc.at[pl.ds(base, RPT)], cnt_out.at[c, pl.ds(base, RPT)])


BLK = 1024


def _mean_agg(sums_ref, cnt_ref):
    cnt = cnt_ref[0] + cnt_ref[1]
    return (sums_ref[0] + sums_ref[1]) / jnp.maximum(cnt, 1.0)


def _l2_normalize(h):
    nrm = jnp.sqrt(jnp.sum(h * h, axis=1, keepdims=True))
    return h / jnp.maximum(nrm, 1e-12)


def _dense1_body(sums_ref, cnt_ref, x_ref, wl_ref, bl_ref, wr_ref, o_ref):
    agg = _mean_agg(sums_ref, cnt_ref)
    h = (jnp.dot(agg, wl_ref[...], preferred_element_type=jnp.float32)
         + jnp.dot(x_ref[...], wr_ref[...], preferred_element_type=jnp.float32)
         + bl_ref[...])
    o_ref[...] = jnp.maximum(_l2_normalize(h), 0.0)


def _dense2_body(sums_ref, cnt_ref, h_ref, wl_ref, bl_ref, wr_ref,
                 wfc_ref, bfc_ref, o_ref):
    agg = _mean_agg(sums_ref, cnt_ref)
    h2 = (jnp.dot(agg, wl_ref[...], preferred_element_type=jnp.float32)
          + jnp.dot(h_ref[...], wr_ref[...], preferred_element_type=jnp.float32)
          + bl_ref[...])
    h2 = _l2_normalize(h2)
    logits = jnp.dot(h2, wfc_ref[...], preferred_element_type=jnp.float32)
    logits = logits + bfc_ref[...]
    m = jnp.max(logits, axis=1, keepdims=True)
    e = jnp.exp(logits - m)
    o_ref[...] = e / jnp.sum(e, axis=1, keepdims=True)


_full = pl.BlockSpec((FEAT, FEAT), lambda i: (0, 0))
_bias = pl.BlockSpec((1, FEAT), lambda i: (0, 0))
_rows = pl.BlockSpec((BLK, FEAT), lambda i: (i, 0))
_sums = pl.BlockSpec((NC, BLK, FEAT), lambda i: (0, i, 0))
_cnts = _sums

_dense1 = pl.pallas_call(
    _dense1_body,
    grid=(N_PAD // BLK,),
    in_specs=[_sums, _cnts, _rows, _full, _bias, _full],
    out_specs=_rows,
    out_shape=jax.ShapeDtypeStruct((N_PAD, FEAT), jnp.float32),
)

_dense2 = pl.pallas_call(
    _dense2_body,
    grid=(N_PAD // BLK,),
    in_specs=[_sums, _cnts, _rows, _full, _bias, _full, _full, _bias],
    out_specs=_rows,
    out_shape=jax.ShapeDtypeStruct((N_PAD, FEAT), jnp.float32),
)


@jax.jit
def kernel(x, edge_index, W1l, b1l, W1r, W2l, b2l, W2r, Wfc, bfc):
    x_pad = jnp.pad(x, ((0, N_PAD - N), (0, 0)))
    pad_src = jnp.arange(E_PAD, dtype=jnp.int32) % N
    pad_dst = N + (jnp.arange(E_PAD, dtype=jnp.int32) % (N_PAD - N))
    src_all = jnp.concatenate([edge_index[0], pad_src])
    dst_all = jnp.concatenate([edge_index[1], pad_dst])
    ec0 = NS * NCH0 * CHUNK
    s0 = src_all[:ec0].reshape(NS, NCH0, CHUNK)
    s1 = jnp.pad(src_all[ec0:].reshape(NS, NCH1, CHUNK),
                 ((0, 0), (0, NCH0 - NCH1), (0, 0)))
    src3 = jnp.concatenate([s0, s1], axis=0)
    d0 = dst_all[:ec0].reshape(NS, NCH0, 1, CHUNK)
    d1 = jnp.pad(dst_all[ec0:].reshape(NS, NCH1, 1, CHUNK),
                 ((0, 0), (0, NCH0 - NCH1), (0, 0), (0, 0)))
    dst4 = jnp.concatenate([d0, d1], axis=0)

    cnt = _sc_counts(dst4)
    sums1 = _sc_agg(x_pad, src3, dst4)
    h1 = _dense1(sums1, cnt, x_pad, W1l.T, b1l.reshape(1, FEAT), W1r.T)

    sums2 = _sc_agg(h1, src3, dst4)
    wfc_pad = jnp.zeros((FEAT, FEAT), jnp.float32).at[:, :NCLASS].set(Wfc.T)
    bfc_pad = jnp.full((1, FEAT), -1e30, jnp.float32).at[0, :NCLASS].set(bfc)
    probs = _dense2(sums2, cnt, h1,
                    W2l.T, b2l.reshape(1, FEAT), W2r.T, wfc_pad, bfc_pad)
    return probs[:N, :NCLASS]


# speedup vs baseline: 3.8098x; 1.0136x over previous
import functools

import jax
import jax.numpy as jnp
from jax import lax
from jax.experimental import pallas as pl
from jax.experimental.pallas import tpu as pltpu
from jax.experimental.pallas import tpu_sc as plsc

N = 10000
E = 320000
FEAT = 128
NCLASS = 40

NC = 2
NS = 16
NLANE = 16
NW = NC * NS
CHUNK = 128
NCH0 = 80
NCH1 = 80
E_TOT = NS * (NCH0 + NCH1) * CHUNK
E_PAD = E_TOT - E
N_PAD = 10240
RPT = N_PAD // NS
CROWS = N_PAD // FEAT
NBUF = 2

_mesh = plsc.VectorSubcoreMesh(core_axis_name="c", subcore_axis_name="s")


@functools.partial(
    pl.kernel,
    out_type=jax.ShapeDtypeStruct((NC, N_PAD, FEAT), jnp.float32),
    mesh=_mesh,
    scratch_types=[
        pltpu.VMEM((NCH0, CHUNK), jnp.int32),
        pltpu.VMEM((NBUF, 1, CHUNK), jnp.int32),
        pltpu.VMEM((NBUF, CHUNK, FEAT), jnp.float32),
        pltpu.VMEM_SHARED((N_PAD, FEAT), jnp.float32),
        pltpu.SemaphoreType.DMA,
        pltpu.SemaphoreType.DMA,
        pltpu.SemaphoreType.DMA,
        pltpu.SemaphoreType.DMA,
    ],
)
def _sc_agg(x_hbm, src_hbm, dst_hbm, sums_out,
            src_v, dst_buf, rows_v, acc, gsem0, gsem1, dsem0, dsem1):
    c = lax.axis_index("c")
    s = lax.axis_index("s")
    wid = c * NS + s
    nch = jnp.where(c == 0, NCH0, NCH1)
    gsems = (gsem0, gsem1)
    dsems = (dsem0, dsem1)

    pltpu.sync_copy(src_hbm.at[wid], src_v)

    zeros16 = jnp.zeros((NLANE,), jnp.float32)

    @pl.loop(0, CHUNK)
    def _(i):
        for j in range(FEAT // NLANE):
            rows_v[0, i, pl.ds(j * NLANE, NLANE)] = zeros16

    base = s * RPT
    for k in range(RPT // CHUNK):
        pltpu.sync_copy(rows_v.at[0], acc.at[pl.ds(base + k * CHUNK, CHUNK)])

    for b in range(NBUF):
        pltpu.async_copy(dst_hbm.at[wid, b], dst_buf.at[b], dsems[b])
        pltpu.async_copy(x_hbm.at[src_v.at[b]], rows_v.at[b], gsems[b])

    plsc.subcore_barrier()

    @pl.loop(0, nch, step=NBUF)
    def _(i0):
        for b in range(NBUF):
            i = i0 + b
            pltpu.make_async_copy(
                dst_hbm.at[wid, i], dst_buf.at[b], dsems[b]).wait()
            pltpu.make_async_copy(
                x_hbm.at[src_v.at[i]], rows_v.at[b], gsems[b]).wait()
            pltpu.sync_copy(rows_v.at[b], acc.at[dst_buf.at[b, 0]], add=True)
            nxt = i + NBUF

            @pl.when(nxt < nch)
            def _():
                pltpu.async_copy(dst_hbm.at[wid, nxt], dst_buf.at[b], dsems[b])
                pltpu.async_copy(
                    x_hbm.at[src_v.at[nxt]], rows_v.at[b], gsems[b])

    plsc.subcore_barrier()
    pltpu.sync_copy(acc.at[pl.ds(base, RPT)], sums_out.at[c, pl.ds(base, RPT)])


@functools.partial(
    pl.kernel,
    out_type=jax.ShapeDtypeStruct((NC, N_PAD, FEAT), jnp.float32),
    mesh=_mesh,
    scratch_types=[
        pltpu.VMEM((NBUF, 1, CHUNK), jnp.int32),
        pltpu.VMEM((CHUNK, FEAT), jnp.float32),
        pltpu.VMEM_SHARED((N_PAD, FEAT), jnp.float32),
        pltpu.SemaphoreType.DMA,
        pltpu.SemaphoreType.DMA,
    ],
)
def _sc_counts(dst_hbm, cnt_out, dst_buf, ones_v, acc, dsem0, dsem1):
    c = lax.axis_index("c")
    s = lax.axis_index("s")
    wid = c * NS + s
    dsems = (dsem0, dsem1)

    for b in range(NBUF):
        pltpu.async_copy(dst_hbm.at[wid, b], dst_buf.at[b], dsems[b])

    zeros16 = jnp.zeros((NLANE,), jnp.float32)
    ones16 = jnp.ones((NLANE,), jnp.float32)

    @pl.loop(0, CHUNK)
    def _(i):
        for j in range(FEAT // NLANE):
            ones_v[i, pl.ds(j * NLANE, NLANE)] = zeros16

    base = s * RPT
    for k in range(RPT // CHUNK):
        pltpu.sync_copy(ones_v, acc.at[pl.ds(base + k * CHUNK, CHUNK)])

    @pl.loop(0, CHUNK)
    def _(i):
        for j in range(FEAT // NLANE):
            ones_v[i, pl.ds(j * NLANE, NLANE)] = ones16

    plsc.subcore_barrier()

    @pl.loop(0, NCH0, step=NBUF)
    def _(i0):
        for b in range(NBUF):
            i = i0 + b
            pltpu.make_async_copy(
                dst_hbm.at[wid, i], dst_buf.at[b], dsems[b]).wait()
            pltpu.sync_copy(ones_v, acc.at[dst_buf.at[b, 0]], add=True)
            nxt = i + NBUF

            @pl.when(nxt < NCH0)
            def _():
                pltpu.async_copy(dst_hbm.at[wid, nxt], dst_buf.at[b], dsems[b])

    plsc.subcore_barrier()
    pltpu.sync_copy(acc.at[pl.ds(base, RPT)], cnt_out.at[c, pl.ds(base, RPT)])


BLK = 1000


def _mean_agg(sums_ref, cnt_ref):
    cnt = cnt_ref[0] + cnt_ref[1]
    return (sums_ref[0] + sums_ref[1]) / jnp.maximum(cnt, 1.0)


def _l2_normalize(h):
    nrm = jnp.sqrt(jnp.sum(h * h, axis=1, keepdims=True))
    return h / jnp.maximum(nrm, 1e-12)


def _dense1_body(sums_ref, cnt_ref, x_ref, wl_ref, bl_ref, wr_ref, o_ref):
    agg = _mean_agg(sums_ref, cnt_ref)
    h = (jnp.dot(agg, wl_ref[...], preferred_element_type=jnp.float32)
         + jnp.dot(x_ref[...], wr_ref[...], preferred_element_type=jnp.float32)
         + bl_ref[...])
    o_ref[...] = jnp.maximum(_l2_normalize(h), 0.0)


def _dense2_body(sums_ref, cnt_ref, h_ref, wl_ref, bl_ref, wr_ref,
                 wfc_ref, bfc_ref, o_ref):
    agg = _mean_agg(sums_ref, cnt_ref)
    h2 = (jnp.dot(agg, wl_ref[...], preferred_element_type=jnp.float32)
          + jnp.dot(h_ref[...], wr_ref[...], preferred_element_type=jnp.float32)
          + bl_ref[...])
    h2 = _l2_normalize(h2)
    logits = jnp.dot(h2, wfc_ref[...], preferred_element_type=jnp.float32)
    logits = logits + bfc_ref[...]
    m = jnp.max(logits, axis=1, keepdims=True)
    e = jnp.exp(logits - m)
    o_ref[...] = e / jnp.sum(e, axis=1, keepdims=True)


_full = pl.BlockSpec((FEAT, FEAT), lambda i: (0, 0))
_bias = pl.BlockSpec((1, FEAT), lambda i: (0, 0))
_rows = pl.BlockSpec((BLK, FEAT), lambda i: (i, 0))
_sums = pl.BlockSpec((NC, BLK, FEAT), lambda i: (0, i, 0))
_cnts = _sums

_dense1 = pl.pallas_call(
    _dense1_body,
    grid=(N // BLK,),
    in_specs=[_sums, _cnts, _rows, _full, _bias, _full],
    out_specs=_rows,
    out_shape=jax.ShapeDtypeStruct((N, FEAT), jnp.float32),
)

_dense2 = pl.pallas_call(
    _dense2_body,
    grid=(N // BLK,),
    in_specs=[_sums, _cnts, _rows, _full, _bias, _full, _full, _bias],
    out_specs=_rows,
    out_shape=jax.ShapeDtypeStruct((N, FEAT), jnp.float32),
)


@jax.jit
def kernel(x, edge_index, W1l, b1l, W1r, W2l, b2l, W2r, Wfc, bfc):
    pad_src = jnp.arange(E_PAD, dtype=jnp.int32) % N
    pad_dst = N + (jnp.arange(E_PAD, dtype=jnp.int32) % (N_PAD - N))
    src_all = jnp.concatenate([edge_index[0], pad_src])
    dst_all = jnp.concatenate([edge_index[1], pad_dst])
    ec0 = NS * NCH0 * CHUNK
    s0 = src_all[:ec0].reshape(NS, NCH0, CHUNK)
    s1 = jnp.pad(src_all[ec0:].reshape(NS, NCH1, CHUNK),
                 ((0, 0), (0, NCH0 - NCH1), (0, 0)))
    src3 = jnp.concatenate([s0, s1], axis=0)
    d0 = dst_all[:ec0].reshape(NS, NCH0, 1, CHUNK)
    d1 = jnp.pad(dst_all[ec0:].reshape(NS, NCH1, 1, CHUNK),
                 ((0, 0), (0, NCH0 - NCH1), (0, 0), (0, 0)))
    dst4 = jnp.concatenate([d0, d1], axis=0)

    sums1 = _sc_agg(x, src3, dst4)
    cnt = _sc_counts(dst4)
    h1 = _dense1(sums1, cnt, x, W1l.T, b1l.reshape(1, FEAT), W1r.T)

    sums2 = _sc_agg(h1, src3, dst4)
    wfc_pad = jnp.zeros((FEAT, FEAT), jnp.float32).at[:, :NCLASS].set(Wfc.T)
    bfc_pad = jnp.full((1, FEAT), -1e30, jnp.float32).at[0, :NCLASS].set(bfc)
    probs = _dense2(sums2, cnt, h1,
                    W2l.T, b2l.reshape(1, FEAT), W2r.T, wfc_pad, bfc_pad)
    return probs[:, :NCLASS]


# speedup vs baseline: 3.8357x; 1.0068x over previous
import functools

import jax
import jax.numpy as jnp
from jax import lax
from jax.experimental import pallas as pl
from jax.experimental.pallas import tpu as pltpu
from jax.experimental.pallas import tpu_sc as plsc

N = 10000
E = 320000
FEAT = 128
NCLASS = 40

NC = 2
NS = 16
NLANE = 16
NW = NC * NS
CHUNK = 128
EPT = 10240
NCHF = EPT // CHUNK
NCHL = (E - (NW - 1) * EPT) // CHUNK
N_PAD = 10240
RPT = N_PAD // NS
CROWS = N_PAD // FEAT
NBUF = 2

_mesh = plsc.VectorSubcoreMesh(core_axis_name="c", subcore_axis_name="s")


@functools.partial(
    pl.kernel,
    out_type=jax.ShapeDtypeStruct((NC, N_PAD, FEAT), jnp.float32),
    mesh=_mesh,
    scratch_types=[
        pltpu.VMEM((EPT,), jnp.int32),
        pltpu.VMEM((NBUF, CHUNK), jnp.int32),
        pltpu.VMEM((NBUF, CHUNK, FEAT), jnp.float32),
        pltpu.VMEM_SHARED((N_PAD, FEAT), jnp.float32),
        pltpu.SemaphoreType.DMA,
        pltpu.SemaphoreType.DMA,
        pltpu.SemaphoreType.DMA,
        pltpu.SemaphoreType.DMA,
    ],
)
def _sc_agg(x_hbm, src_hbm, dst_hbm, sums_out,
            src_v, dst_buf, rows_v, acc, gsem0, gsem1, dsem0, dsem1):
    c = lax.axis_index("c")
    s = lax.axis_index("s")
    wid = c * NS + s
    ebase = wid * EPT
    nch = jnp.where(wid == NW - 1, NCHL, NCHF)
    gsems = (gsem0, gsem1)
    dsems = (dsem0, dsem1)

    @pl.when(wid < NW - 1)
    def _():
        pltpu.sync_copy(src_hbm.at[pl.ds(ebase, EPT)], src_v)

    @pl.when(wid == NW - 1)
    def _():
        pltpu.sync_copy(src_hbm.at[pl.ds(ebase, NCHL * CHUNK)],
                        src_v.at[pl.ds(0, NCHL * CHUNK)])

    zeros16 = jnp.zeros((NLANE,), jnp.float32)

    @pl.loop(0, CHUNK)
    def _(i):
        for j in range(FEAT // NLANE):
            rows_v[0, i, pl.ds(j * NLANE, NLANE)] = zeros16

    base = s * RPT
    for k in range(RPT // CHUNK):
        pltpu.sync_copy(rows_v.at[0], acc.at[pl.ds(base + k * CHUNK, CHUNK)])

    for b in range(NBUF):
        pltpu.async_copy(dst_hbm.at[pl.ds(ebase + b * CHUNK, CHUNK)],
                         dst_buf.at[b], dsems[b])
        pltpu.async_copy(x_hbm.at[src_v.at[pl.ds(b * CHUNK, CHUNK)]],
                         rows_v.at[b], gsems[b])

    plsc.subcore_barrier()

    @pl.loop(0, nch, step=NBUF)
    def _(i0):
        for b in range(NBUF):
            i = i0 + b
            off = pl.multiple_of(i * CHUNK, CHUNK)
            pltpu.make_async_copy(
                dst_hbm.at[pl.ds(ebase + off, CHUNK)],
                dst_buf.at[b], dsems[b]).wait()
            pltpu.make_async_copy(
                x_hbm.at[src_v.at[pl.ds(off, CHUNK)]],
                rows_v.at[b], gsems[b]).wait()
            pltpu.sync_copy(rows_v.at[b], acc.at[dst_buf.at[b]], add=True)
            nxt = i + NBUF

            @pl.when(nxt < nch)
            def _():
                noff = pl.multiple_of(nxt * CHUNK, CHUNK)
                pltpu.async_copy(dst_hbm.at[pl.ds(ebase + noff, CHUNK)],
                                 dst_buf.at[b], dsems[b])
                pltpu.async_copy(x_hbm.at[src_v.at[pl.ds(noff, CHUNK)]],
                                 rows_v.at[b], gsems[b])

    plsc.subcore_barrier()
    pltpu.sync_copy(acc.at[pl.ds(base, RPT)], sums_out.at[c, pl.ds(base, RPT)])


@functools.partial(
    pl.kernel,
    out_type=jax.ShapeDtypeStruct((NC, N_PAD, FEAT), jnp.float32),
    mesh=_mesh,
    scratch_types=[
        pltpu.VMEM((NBUF, CHUNK), jnp.int32),
        pltpu.VMEM((CHUNK, FEAT), jnp.float32),
        pltpu.VMEM_SHARED((N_PAD, FEAT), jnp.float32),
        pltpu.SemaphoreType.DMA,
        pltpu.SemaphoreType.DMA,
    ],
)
def _sc_counts(dst_hbm, cnt_out, dst_buf, ones_v, acc, dsem0, dsem1):
    c = lax.axis_index("c")
    s = lax.axis_index("s")
    wid = c * NS + s
    ebase = wid * EPT
    nch = jnp.where(wid == NW - 1, NCHL, NCHF)
    dsems = (dsem0, dsem1)

    for b in range(NBUF):
        pltpu.async_copy(dst_hbm.at[pl.ds(ebase + b * CHUNK, CHUNK)],
                         dst_buf.at[b], dsems[b])

    zeros16 = jnp.zeros((NLANE,), jnp.float32)
    ones16 = jnp.ones((NLANE,), jnp.float32)

    @pl.loop(0, CHUNK)
    def _(i):
        for j in range(FEAT // NLANE):
            ones_v[i, pl.ds(j * NLANE, NLANE)] = zeros16

    base = s * RPT
    for k in range(RPT // CHUNK):
        pltpu.sync_copy(ones_v, acc.at[pl.ds(base + k * CHUNK, CHUNK)])

    @pl.loop(0, CHUNK)
    def _(i):
        for j in range(FEAT // NLANE):
            ones_v[i, pl.ds(j * NLANE, NLANE)] = ones16

    plsc.subcore_barrier()

    @pl.loop(0, nch, step=NBUF)
    def _(i0):
        for b in range(NBUF):
            i = i0 + b
            off = pl.multiple_of(i * CHUNK, CHUNK)
            pltpu.make_async_copy(
                dst_hbm.at[pl.ds(ebase + off, CHUNK)],
                dst_buf.at[b], dsems[b]).wait()
            pltpu.sync_copy(ones_v, acc.at[dst_buf.at[b]], add=True)
            nxt = i + NBUF

            @pl.when(nxt < nch)
            def _():
                noff = pl.multiple_of(nxt * CHUNK, CHUNK)
                pltpu.async_copy(dst_hbm.at[pl.ds(ebase + noff, CHUNK)],
                                 dst_buf.at[b], dsems[b])

    plsc.subcore_barrier()
    pltpu.sync_copy(acc.at[pl.ds(base, RPT)], cnt_out.at[c, pl.ds(base, RPT)])


BLK = 1000


def _mean_agg(sums_ref, cnt_ref):
    cnt = cnt_ref[0] + cnt_ref[1]
    return (sums_ref[0] + sums_ref[1]) / jnp.maximum(cnt, 1.0)


def _l2_normalize(h):
    nrm = jnp.sqrt(jnp.sum(h * h, axis=1, keepdims=True))
    return h / jnp.maximum(nrm, 1e-12)


def _dense1_body(sums_ref, cnt_ref, x_ref, wl_ref, bl_ref, wr_ref, o_ref):
    agg = _mean_agg(sums_ref, cnt_ref)
    h = (jnp.dot(agg, wl_ref[...], preferred_element_type=jnp.float32)
         + jnp.dot(x_ref[...], wr_ref[...], preferred_element_type=jnp.float32)
         + bl_ref[...])
    o_ref[...] = jnp.maximum(_l2_normalize(h), 0.0)


def _dense2_body(sums_ref, cnt_ref, h_ref, wl_ref, bl_ref, wr_ref,
                 wfc_ref, bfc_ref, o_ref):
    agg = _mean_agg(sums_ref, cnt_ref)
    h2 = (jnp.dot(agg, wl_ref[...], preferred_element_type=jnp.float32)
          + jnp.dot(h_ref[...], wr_ref[...], preferred_element_type=jnp.float32)
          + bl_ref[...])
    h2 = _l2_normalize(h2)
    logits = jnp.dot(h2, wfc_ref[...], preferred_element_type=jnp.float32)
    logits = logits + bfc_ref[...]
    m = jnp.max(logits, axis=1, keepdims=True)
    e = jnp.exp(logits - m)
    o_ref[...] = e / jnp.sum(e, axis=1, keepdims=True)


_full = pl.BlockSpec((FEAT, FEAT), lambda i: (0, 0))
_bias = pl.BlockSpec((1, FEAT), lambda i: (0, 0))
_rows = pl.BlockSpec((BLK, FEAT), lambda i: (i, 0))
_sums = pl.BlockSpec((NC, BLK, FEAT), lambda i: (0, i, 0))
_cnts = _sums

_dense1 = pl.pallas_call(
    _dense1_body,
    grid=(N // BLK,),
    in_specs=[_sums, _cnts, _rows, _full, _bias, _full],
    out_specs=_rows,
    out_shape=jax.ShapeDtypeStruct((N, FEAT), jnp.float32),
)

_dense2 = pl.pallas_call(
    _dense2_body,
    grid=(N // BLK,),
    in_specs=[_sums, _cnts, _rows, _full, _bias, _full, _full, _bias],
    out_specs=_rows,
    out_shape=jax.ShapeDtypeStruct((N, FEAT), jnp.float32),
)


@jax.jit
def kernel(x, edge_index, W1l, b1l, W1r, W2l, b2l, W2r, Wfc, bfc):
    src = edge_index[0]
    dst = edge_index[1]

    sums1 = _sc_agg(x, src, dst)
    cnt = _sc_counts(dst)
    h1 = _dense1(sums1, cnt, x, W1l.T, b1l.reshape(1, FEAT), W1r.T)

    sums2 = _sc_agg(h1, src, dst)
    wfc_pad = jnp.zeros((FEAT, FEAT), jnp.float32).at[:, :NCLASS].set(Wfc.T)
    bfc_pad = jnp.full((1, FEAT), -1e30, jnp.float32).at[0, :NCLASS].set(bfc)
    probs = _dense2(sums2, cnt, h1,
                    W2l.T, b2l.reshape(1, FEAT), W2r.T, wfc_pad, bfc_pad)
    return probs[:, :NCLASS]
